# baseline (speedup 1.0000x reference)
import functools

import jax
import jax.numpy as jnp
from jax import lax
from jax.experimental import pallas as pl
from jax.experimental.pallas import tpu as pltpu
from jax.experimental.pallas import tpu_sc as plsc

N = 10000
E = 320000
D = 128

NC = 2
NS = 16
LANES = 16

K = 128
CPP = 160
CH0 = CPP // 2
CH1 = CPP - CH0
CHMAX = max(CH0, CH1)
EPW = CPP * K
EPAD = NS * EPW - E
NPAD = 10240
ROWS_PS = NPAD // NS
RBLK = 2048
GRID = NPAD // RBLK

_mesh = plsc.VectorSubcoreMesh(core_axis_name="c", subcore_axis_name="s")

_F32 = jnp.float32
_I32 = jnp.int32
_PREC = jax.lax.Precision.HIGHEST


def _unpack(pidx, j, sidx_s, didx_s):
    for i in range(K // LANES):
        v = pidx[j, pl.ds(i * LANES, LANES)]
        sidx_s[pl.ds(i * LANES, LANES)] = lax.bitwise_and(v, 0xFFFF)
        didx_s[pl.ds(i * LANES, LANES)] = lax.shift_right_logical(v, 16)


@functools.partial(
    pl.kernel,
    out_type=[jax.ShapeDtypeStruct((NPAD,), _F32),
              jax.ShapeDtypeStruct((NPAD,), _F32)],
    mesh=_mesh,
    scratch_types=[
        pltpu.VMEM_SHARED((NPAD,), _F32),
        pltpu.VMEM((CPP // 2, K), _I32),
        pltpu.VMEM((K,), _I32),
        pltpu.VMEM((K,), _I32),
        pltpu.VMEM((K,), _F32),
        pltpu.VMEM((ROWS_PS,), _F32),
    ],
)
def _sc_degree(pidx_hbm, out0_hbm, out1_hbm, deg_sh, pidx, didx_s, sidx_s,
               ones_v, zeros_v):
    c = lax.axis_index("c")
    s = lax.axis_index("s")

    @pl.loop(0, ROWS_PS // LANES)
    def _(i):
        zeros_v[pl.ds(i * LANES, LANES)] = jnp.zeros((LANES,), _F32)

    @pl.loop(0, K // LANES)
    def _(i):
        ones_v[pl.ds(i * LANES, LANES)] = jnp.ones((LANES,), _F32)

    pltpu.sync_copy(zeros_v, deg_sh.at[pl.ds(s * ROWS_PS, ROWS_PS)])
    plsc.subcore_barrier()

    pltpu.sync_copy(pidx_hbm.at[s].at[pl.ds(c * (CPP // 2), CPP // 2)], pidx)

    @pl.loop(0, CPP // 2)
    def _(j):
        _unpack(pidx, j, sidx_s, didx_s)
        pltpu.sync_copy(ones_v, deg_sh.at[didx_s], add=True)

    plsc.subcore_barrier()

    @pl.when(c == 0)
    def _():
        pltpu.sync_copy(deg_sh.at[pl.ds(s * ROWS_PS, ROWS_PS)],
                        out0_hbm.at[pl.ds(s * ROWS_PS, ROWS_PS)])

    @pl.when(c == 1)
    def _():
        pltpu.sync_copy(deg_sh.at[pl.ds(s * ROWS_PS, ROWS_PS)],
                        out1_hbm.at[pl.ds(s * ROWS_PS, ROWS_PS)])


@functools.partial(
    pl.kernel,
    out_type=jax.ShapeDtypeStruct((NC, NPAD, D), _F32),
    mesh=_mesh,
    scratch_types=[
        pltpu.VMEM_SHARED((NPAD, D), _F32),
        pltpu.VMEM((CHMAX, K), _I32),
        pltpu.VMEM((K,), _I32),
        pltpu.VMEM((K,), _I32),
        pltpu.VMEM((K,), _I32),
        pltpu.VMEM((K,), _I32),
        pltpu.VMEM((K, D), _F32),
        pltpu.VMEM((K, D), _F32),
        pltpu.SemaphoreType.DMA,
        pltpu.SemaphoreType.DMA,
    ],
)
def _sc_aggregate(hs_hbm, pidx_hbm, out_hbm, agg_sh, pidx, sidx_a, didx_a,
                  sidx_b, didx_b, rows_a, rows_b, sem_a, sem_b):
    c = lax.axis_index("c")
    s = lax.axis_index("s")

    @pl.loop(0, (K * D) // LANES)
    def _(t):
        r = t // (D // LANES)
        q = t % (D // LANES)
        rows_a[r, pl.ds(q * LANES, LANES)] = jnp.zeros((LANES,), _F32)

    @pl.loop(0, ROWS_PS // K)
    def _(i):
        pltpu.sync_copy(rows_a, agg_sh.at[pl.ds(s * ROWS_PS + i * K, K)])

    plsc.subcore_barrier()

    def pipeline(n):
        @pl.loop(0, n)
        def _(j):
            _unpack(pidx, j, sidx_a, didx_a)
            pltpu.async_copy(hs_hbm.at[sidx_a], rows_a, sem_a).wait()
            pltpu.sync_copy(rows_a, agg_sh.at[didx_a], add=True)

    @pl.when(c == 0)
    def _():
        pltpu.sync_copy(pidx_hbm.at[s].at[pl.ds(0, CH0)],
                        pidx.at[pl.ds(0, CH0)])
        pipeline(CH0)

    @pl.when(c == 1)
    def _():
        pltpu.sync_copy(pidx_hbm.at[s].at[pl.ds(CH0, CH1)],
                        pidx.at[pl.ds(0, CH1)])
        pipeline(CH1)

    plsc.subcore_barrier()

    @pl.loop(0, ROWS_PS // K)
    def _(i):
        pltpu.sync_copy(agg_sh.at[pl.ds(s * ROWS_PS + i * K, K)],
                        out_hbm.at[c].at[pl.ds(s * ROWS_PS + i * K, K)])


def _pre_body(x_ref, w_ref, b_ref, dis_ref, h_ref, hs_ref):
    h = jnp.dot(x_ref[...], w_ref[...], precision=_PREC,
                preferred_element_type=_F32)
    h = jnp.maximum(h + b_ref[...], 0.0)
    h_ref[...] = h
    hs_ref[...] = h * dis_ref[...]


def _mid_body(aggp_ref, dis_ref, h_ref, w_ref, b_ref, hn_ref, hs_ref):
    agg = (aggp_ref[0] + aggp_ref[1]) * dis_ref[...]
    hn = jnp.dot(agg, w_ref[...], precision=_PREC, preferred_element_type=_F32)
    hn = jnp.maximum(hn + b_ref[...] + h_ref[...], 0.0)
    hn_ref[...] = hn
    hs_ref[...] = hn * dis_ref[...]


def _final_body(aggp_ref, dis_ref, h_ref, w_ref, b_ref, wh_ref, bh_ref,
                out_ref):
    agg = (aggp_ref[0] + aggp_ref[1]) * dis_ref[...]
    h3 = jnp.dot(agg, w_ref[...], precision=_PREC, preferred_element_type=_F32)
    h3 = jnp.maximum(h3 + b_ref[...] + h_ref[...], 0.0)
    out_ref[...] = jnp.dot(h3, wh_ref[...], precision=_PREC,
                           preferred_element_type=_F32) + bh_ref[...]


_row_spec = pl.BlockSpec((RBLK, D), lambda i: (i, 0))
_mat_spec = pl.BlockSpec((D, D), lambda i: (0, 0))
_bias_spec = pl.BlockSpec((1, D), lambda i: (0, 0))
_aggp_spec = pl.BlockSpec((NC, RBLK, D), lambda i: (0, i, 0))
_nd_shape = jax.ShapeDtypeStruct((NPAD, D), _F32)


_tc_pre = pl.pallas_call(
    _pre_body,
    grid=(GRID,),
    in_specs=[_row_spec, _mat_spec, _bias_spec, _row_spec],
    out_specs=[_row_spec, _row_spec],
    out_shape=[_nd_shape, _nd_shape],
)

_tc_mid = pl.pallas_call(
    _mid_body,
    grid=(GRID,),
    in_specs=[_aggp_spec, _row_spec, _row_spec, _mat_spec, _bias_spec],
    out_specs=[_row_spec, _row_spec],
    out_shape=[_nd_shape, _nd_shape],
)

_tc_final = pl.pallas_call(
    _final_body,
    grid=(GRID,),
    in_specs=[_aggp_spec, _row_spec, _row_spec, _mat_spec, _bias_spec,
              _mat_spec, _bias_spec],
    out_specs=_row_spec,
    out_shape=_nd_shape,
)


def kernel(x, edge_index, W_pre, b_pre, W1, b1, W2, b2, W3, b3, W_head,
           b_head):
    src_p = jnp.concatenate([edge_index[0], jnp.zeros((EPAD,), _I32)])
    pad_dst = N + (jnp.arange(EPAD, dtype=_I32) % (NPAD - N))
    dst_p = jnp.concatenate([edge_index[1], pad_dst])
    pidx = (src_p | (dst_p << 16)).reshape(NS, CPP, K)

    deg0, deg1 = _sc_degree(pidx)
    dis = jax.lax.rsqrt(jnp.maximum(deg0 + deg1, 1.0))
    dis_full = jnp.broadcast_to(dis[:, None], (NPAD, D))

    x_p = jnp.pad(x, ((0, NPAD - N), (0, 0)))
    h, hs = _tc_pre(x_p, W_pre, b_pre.reshape(1, D), dis_full)

    for W, b in ((W1, b1), (W2, b2)):
        aggp = _sc_aggregate(hs, pidx)
        h, hs = _tc_mid(aggp, dis_full, h, W, b.reshape(1, D))

    aggp = _sc_aggregate(hs, pidx)
    out = _tc_final(aggp, dis_full, h, W3, b3.reshape(1, D), W_head,
                    b_head.reshape(1, D))
    return out[:N]


# speedup vs baseline: 1.1150x; 1.1150x over previous
import functools

import jax
import jax.numpy as jnp
from jax import lax
from jax.experimental import pallas as pl
from jax.experimental.pallas import tpu as pltpu
from jax.experimental.pallas import tpu_sc as plsc

N = 10000
E = 320000
D = 128

NC = 2
NS = 16
LANES = 16

K = 80
CPP = 256
CH = CPP // 2
EPW = CPP * K
EPAD = NS * EPW - E
NPAD = 10240
ROWS_PS = NPAD // NS
RBLK = 2048
GRID = NPAD // RBLK

_mesh = plsc.VectorSubcoreMesh(core_axis_name="c", subcore_axis_name="s")

_F32 = jnp.float32
_I32 = jnp.int32
_PREC = jax.lax.Precision.HIGHEST


def _unpack(pidx, j, sidx_s, didx_s):
    for i in range(K // LANES):
        v = pidx[j, pl.ds(i * LANES, LANES)]
        sidx_s[0, pl.ds(i * LANES, LANES)] = lax.bitwise_and(v, 0xFFFF)
        didx_s[0, pl.ds(i * LANES, LANES)] = lax.shift_right_logical(v, 16)


@functools.partial(
    pl.kernel,
    out_type=[jax.ShapeDtypeStruct((NPAD,), _F32),
              jax.ShapeDtypeStruct((NPAD,), _F32)],
    mesh=_mesh,
    scratch_types=[
        pltpu.VMEM_SHARED((NPAD,), _F32),
        pltpu.VMEM((CH, K), _I32),
        pltpu.VMEM((1, K), _I32),
        pltpu.VMEM((1, K), _I32),
        pltpu.VMEM((K,), _F32),
        pltpu.VMEM((ROWS_PS,), _F32),
    ],
)
def _sc_degree(pidx_hbm, out0_hbm, out1_hbm, deg_sh, pidx, sidx_s, didx_s,
               ones_v, zeros_v):
    c = lax.axis_index("c")
    s = lax.axis_index("s")

    @pl.loop(0, ROWS_PS // LANES)
    def _(i):
        zeros_v[pl.ds(i * LANES, LANES)] = jnp.zeros((LANES,), _F32)

    @pl.loop(0, K // LANES)
    def _(i):
        ones_v[pl.ds(i * LANES, LANES)] = jnp.ones((LANES,), _F32)

    pltpu.sync_copy(zeros_v, deg_sh.at[pl.ds(s * ROWS_PS, ROWS_PS)])
    plsc.subcore_barrier()

    pltpu.sync_copy(pidx_hbm.at[s].at[pl.ds(c * CH, CH)], pidx)

    @pl.loop(0, CH)
    def _(j):
        _unpack(pidx, j, sidx_s, didx_s)
        pltpu.sync_copy(ones_v, deg_sh.at[didx_s.at[0]], add=True)

    plsc.subcore_barrier()

    @pl.when(c == 0)
    def _():
        pltpu.sync_copy(deg_sh.at[pl.ds(s * ROWS_PS, ROWS_PS)],
                        out0_hbm.at[pl.ds(s * ROWS_PS, ROWS_PS)])

    @pl.when(c == 1)
    def _():
        pltpu.sync_copy(deg_sh.at[pl.ds(s * ROWS_PS, ROWS_PS)],
                        out1_hbm.at[pl.ds(s * ROWS_PS, ROWS_PS)])


@functools.partial(
    pl.kernel,
    out_type=jax.ShapeDtypeStruct((NC, NPAD, D), _F32),
    mesh=_mesh,
    scratch_types=[
        pltpu.VMEM_SHARED((NPAD, D), _F32),
        pltpu.VMEM((CH, K), _I32),
        pltpu.VMEM((1, K), _I32),
        pltpu.VMEM((1, K), _I32),
        pltpu.VMEM((1, K), _I32),
        pltpu.VMEM((1, K), _I32),
        pltpu.VMEM((K, D), _F32),
        pltpu.VMEM((K, D), _F32),
        pltpu.SemaphoreType.DMA,
        pltpu.SemaphoreType.DMA,
    ],
)
def _sc_aggregate(hs_hbm, pidx_hbm, out_hbm, agg_sh, pidx, sidx_a, didx_a,
                  sidx_b, didx_b, rows_a, rows_b, sem_a, sem_b):
    c = lax.axis_index("c")
    s = lax.axis_index("s")

    @pl.loop(0, (K * D) // LANES)
    def _(t):
        r = t // (D // LANES)
        q = t % (D // LANES)
        rows_a[r, pl.ds(q * LANES, LANES)] = jnp.zeros((LANES,), _F32)

    @pl.loop(0, ROWS_PS // K)
    def _(i):
        pltpu.sync_copy(rows_a, agg_sh.at[pl.ds(s * ROWS_PS + i * K, K)])

    plsc.subcore_barrier()

    pltpu.sync_copy(pidx_hbm.at[s].at[pl.ds(c * CH, CH)], pidx)

    _unpack(pidx, 0, sidx_a, didx_a)
    pltpu.async_copy(hs_hbm.at[sidx_a.at[0]], rows_a, sem_a)

    @pl.loop(0, CH - 2, step=2)
    def _(j):
        _unpack(pidx, j + 1, sidx_b, didx_b)
        pltpu.async_copy(hs_hbm.at[sidx_b.at[0]], rows_b, sem_b)
        pltpu.make_async_copy(hs_hbm.at[sidx_a.at[0]], rows_a, sem_a).wait()
        pltpu.sync_copy(rows_a, agg_sh.at[didx_a.at[0]], add=True)
        _unpack(pidx, j + 2, sidx_a, didx_a)
        pltpu.async_copy(hs_hbm.at[sidx_a.at[0]], rows_a, sem_a)
        pltpu.make_async_copy(hs_hbm.at[sidx_b.at[0]], rows_b, sem_b).wait()
        pltpu.sync_copy(rows_b, agg_sh.at[didx_b.at[0]], add=True)

    _unpack(pidx, CH - 1, sidx_b, didx_b)
    pltpu.async_copy(hs_hbm.at[sidx_b.at[0]], rows_b, sem_b)
    pltpu.make_async_copy(hs_hbm.at[sidx_a.at[0]], rows_a, sem_a).wait()
    pltpu.sync_copy(rows_a, agg_sh.at[didx_a.at[0]], add=True)
    pltpu.make_async_copy(hs_hbm.at[sidx_b.at[0]], rows_b, sem_b).wait()
    pltpu.sync_copy(rows_b, agg_sh.at[didx_b.at[0]], add=True)

    plsc.subcore_barrier()

    @pl.loop(0, ROWS_PS // K)
    def _(i):
        pltpu.sync_copy(agg_sh.at[pl.ds(s * ROWS_PS + i * K, K)],
                        out_hbm.at[c].at[pl.ds(s * ROWS_PS + i * K, K)])


def _pre_body(x_ref, w_ref, b_ref, dis_ref, h_ref, hs_ref):
    h = jnp.dot(x_ref[...], w_ref[...], precision=_PREC,
                preferred_element_type=_F32)
    h = jnp.maximum(h + b_ref[...], 0.0)
    h_ref[...] = h
    hs_ref[...] = h * dis_ref[...]


def _mid_body(aggp_ref, dis_ref, h_ref, w_ref, b_ref, hn_ref, hs_ref):
    agg = (aggp_ref[0] + aggp_ref[1]) * dis_ref[...]
    hn = jnp.dot(agg, w_ref[...], precision=_PREC, preferred_element_type=_F32)
    hn = jnp.maximum(hn + b_ref[...] + h_ref[...], 0.0)
    hn_ref[...] = hn
    hs_ref[...] = hn * dis_ref[...]


def _final_body(aggp_ref, dis_ref, h_ref, w_ref, b_ref, wh_ref, bh_ref,
                out_ref):
    agg = (aggp_ref[0] + aggp_ref[1]) * dis_ref[...]
    h3 = jnp.dot(agg, w_ref[...], precision=_PREC, preferred_element_type=_F32)
    h3 = jnp.maximum(h3 + b_ref[...] + h_ref[...], 0.0)
    out_ref[...] = jnp.dot(h3, wh_ref[...], precision=_PREC,
                           preferred_element_type=_F32) + bh_ref[...]


_row_spec = pl.BlockSpec((RBLK, D), lambda i: (i, 0))
_mat_spec = pl.BlockSpec((D, D), lambda i: (0, 0))
_bias_spec = pl.BlockSpec((1, D), lambda i: (0, 0))
_aggp_spec = pl.BlockSpec((NC, RBLK, D), lambda i: (0, i, 0))
_nd_shape = jax.ShapeDtypeStruct((NPAD, D), _F32)


_tc_pre = pl.pallas_call(
    _pre_body,
    grid=(GRID,),
    in_specs=[_row_spec, _mat_spec, _bias_spec, _row_spec],
    out_specs=[_row_spec, _row_spec],
    out_shape=[_nd_shape, _nd_shape],
)

_tc_mid = pl.pallas_call(
    _mid_body,
    grid=(GRID,),
    in_specs=[_aggp_spec, _row_spec, _row_spec, _mat_spec, _bias_spec],
    out_specs=[_row_spec, _row_spec],
    out_shape=[_nd_shape, _nd_shape],
)

_tc_final = pl.pallas_call(
    _final_body,
    grid=(GRID,),
    in_specs=[_aggp_spec, _row_spec, _row_spec, _mat_spec, _bias_spec,
              _mat_spec, _bias_spec],
    out_specs=_row_spec,
    out_shape=_nd_shape,
)


def kernel(x, edge_index, W_pre, b_pre, W1, b1, W2, b2, W3, b3, W_head,
           b_head):
    src_p = jnp.concatenate([edge_index[0], jnp.zeros((EPAD,), _I32)])
    pad_dst = N + (jnp.arange(EPAD, dtype=_I32) % (NPAD - N))
    dst_p = jnp.concatenate([edge_index[1], pad_dst])
    pidx = (src_p | (dst_p << 16)).reshape(NS, CPP, K)

    deg0, deg1 = _sc_degree(pidx)
    dis = jax.lax.rsqrt(jnp.maximum(deg0 + deg1, 1.0))
    dis_full = jnp.broadcast_to(dis[:, None], (NPAD, D))

    x_p = jnp.pad(x, ((0, NPAD - N), (0, 0)))
    h, hs = _tc_pre(x_p, W_pre, b_pre.reshape(1, D), dis_full)

    for W, b in ((W1, b1), (W2, b2)):
        aggp = _sc_aggregate(hs, pidx)
        h, hs = _tc_mid(aggp, dis_full, h, W, b.reshape(1, D))

    aggp = _sc_aggregate(hs, pidx)
    out = _tc_final(aggp, dis_full, h, W3, b3.reshape(1, D), W_head,
                    b_head.reshape(1, D))
    return out[:N]


# speedup vs baseline: 1.8676x; 1.6751x over previous
import functools

import jax
import jax.numpy as jnp
from jax import lax
from jax.experimental import pallas as pl
from jax.experimental.pallas import tpu as pltpu
from jax.experimental.pallas import tpu_sc as plsc

N = 10000
E = 320000
D = 128

NC = 2
NS = 16
NW = NC * NS
LANES = 16

K = 120
CHUNKS = 84
EPW = CHUNKS * K
EPAD = NW * EPW - E
NPAD = 10240
ROWS_PS = NPAD // NS
WB = 80
RBLK = 2048
GRID = NPAD // RBLK

_mesh = plsc.VectorSubcoreMesh(core_axis_name="c", subcore_axis_name="s")

_F32 = jnp.float32
_I32 = jnp.int32
_PREC = jax.lax.Precision.HIGHEST


@functools.partial(
    pl.kernel,
    out_type=[jax.ShapeDtypeStruct((NPAD,), _F32),
              jax.ShapeDtypeStruct((NPAD,), _F32)],
    mesh=_mesh,
    scratch_types=[
        pltpu.VMEM_SHARED((NPAD,), _F32),
        pltpu.VMEM((CHUNKS, K), jnp.int32),
        pltpu.VMEM((128,), _F32),
        pltpu.VMEM((ROWS_PS,), _F32),
    ],
)
def _sc_degree(dst_hbm, out0_hbm, out1_hbm, deg_sh, didx, ones_v, zeros_v):
    c = lax.axis_index("c")
    s = lax.axis_index("s")
    wid = s * NC + c

    @pl.loop(0, ROWS_PS // LANES)
    def _(i):
        zeros_v[pl.ds(i * LANES, LANES)] = jnp.zeros((LANES,), _F32)

    @pl.loop(0, 128 // LANES)
    def _(i):
        ones_v[pl.ds(i * LANES, LANES)] = jnp.ones((LANES,), _F32)

    pltpu.sync_copy(zeros_v, deg_sh.at[pl.ds(s * ROWS_PS, ROWS_PS)])
    plsc.subcore_barrier()

    pltpu.sync_copy(dst_hbm.at[wid], didx)

    @pl.loop(0, CHUNKS)
    def _(j):
        pltpu.sync_copy(ones_v.at[pl.ds(0, K)], deg_sh.at[didx.at[j]],
                        add=True)

    plsc.subcore_barrier()

    @pl.when(c == 0)
    def _():
        pltpu.sync_copy(deg_sh.at[pl.ds(s * ROWS_PS, ROWS_PS)],
                        out0_hbm.at[pl.ds(s * ROWS_PS, ROWS_PS)])

    @pl.when(c == 1)
    def _():
        pltpu.sync_copy(deg_sh.at[pl.ds(s * ROWS_PS, ROWS_PS)],
                        out1_hbm.at[pl.ds(s * ROWS_PS, ROWS_PS)])


@functools.partial(
    pl.kernel,
    out_type=jax.ShapeDtypeStruct((NC, NPAD, D), _F32),
    mesh=_mesh,
    scratch_types=[
        pltpu.VMEM_SHARED((NPAD, D), _F32),
        pltpu.VMEM((CHUNKS, K), jnp.int32),
        pltpu.VMEM((CHUNKS, K), jnp.int32),
        pltpu.VMEM((K, D), _F32),
        pltpu.SemaphoreType.DMA,
    ],
)
def _sc_aggregate(hs_hbm, src_hbm, dst_hbm, out_hbm, agg_sh, sidx, didx,
                  rows, sem):
    c = lax.axis_index("c")
    s = lax.axis_index("s")
    wid = s * NC + c

    @pl.loop(0, (WB * D) // LANES)
    def _(t):
        r = t // (D // LANES)
        q = t % (D // LANES)
        rows[r, pl.ds(q * LANES, LANES)] = jnp.zeros((LANES,), _F32)

    @pl.loop(0, ROWS_PS // WB)
    def _(i):
        pltpu.sync_copy(rows.at[pl.ds(0, WB)],
                        agg_sh.at[pl.ds(s * ROWS_PS + i * WB, WB)])

    plsc.subcore_barrier()

    pltpu.sync_copy(src_hbm.at[wid], sidx)
    pltpu.sync_copy(dst_hbm.at[wid], didx)

    @pl.loop(0, CHUNKS)
    def _(j):
        pltpu.async_copy(hs_hbm.at[sidx.at[j]], rows, sem).wait()
        pltpu.sync_copy(rows, agg_sh.at[didx.at[j]], add=True)

    plsc.subcore_barrier()

    @pl.loop(0, ROWS_PS // WB)
    def _(i):
        pltpu.sync_copy(agg_sh.at[pl.ds(s * ROWS_PS + i * WB, WB)],
                        out_hbm.at[c].at[pl.ds(s * ROWS_PS + i * WB, WB)])


def _pre_body(x_ref, w_ref, b_ref, dis_ref, h_ref, hs_ref):
    h = jnp.dot(x_ref[...], w_ref[...], precision=_PREC,
                preferred_element_type=_F32)
    h = jnp.maximum(h + b_ref[...], 0.0)
    h_ref[...] = h
    hs_ref[...] = h * dis_ref[...]


def _mid_body(aggp_ref, dis_ref, h_ref, w_ref, b_ref, hn_ref, hs_ref):
    agg = (aggp_ref[0] + aggp_ref[1]) * dis_ref[...]
    hn = jnp.dot(agg, w_ref[...], precision=_PREC, preferred_element_type=_F32)
    hn = jnp.maximum(hn + b_ref[...] + h_ref[...], 0.0)
    hn_ref[...] = hn
    hs_ref[...] = hn * dis_ref[...]


def _final_body(aggp_ref, dis_ref, h_ref, w_ref, b_ref, wh_ref, bh_ref,
                out_ref):
    agg = (aggp_ref[0] + aggp_ref[1]) * dis_ref[...]
    h3 = jnp.dot(agg, w_ref[...], precision=_PREC, preferred_element_type=_F32)
    h3 = jnp.maximum(h3 + b_ref[...] + h_ref[...], 0.0)
    out_ref[...] = jnp.dot(h3, wh_ref[...], precision=_PREC,
                           preferred_element_type=_F32) + bh_ref[...]


_row_spec = pl.BlockSpec((RBLK, D), lambda i: (i, 0))
_mat_spec = pl.BlockSpec((D, D), lambda i: (0, 0))
_bias_spec = pl.BlockSpec((1, D), lambda i: (0, 0))
_aggp_spec = pl.BlockSpec((NC, RBLK, D), lambda i: (0, i, 0))
_nd_shape = jax.ShapeDtypeStruct((NPAD, D), _F32)


_tc_pre = pl.pallas_call(
    _pre_body,
    grid=(GRID,),
    in_specs=[_row_spec, _mat_spec, _bias_spec, _row_spec],
    out_specs=[_row_spec, _row_spec],
    out_shape=[_nd_shape, _nd_shape],
)

_tc_mid = pl.pallas_call(
    _mid_body,
    grid=(GRID,),
    in_specs=[_aggp_spec, _row_spec, _row_spec, _mat_spec, _bias_spec],
    out_specs=[_row_spec, _row_spec],
    out_shape=[_nd_shape, _nd_shape],
)

_tc_final = pl.pallas_call(
    _final_body,
    grid=(GRID,),
    in_specs=[_aggp_spec, _row_spec, _row_spec, _mat_spec, _bias_spec,
              _mat_spec, _bias_spec],
    out_specs=_row_spec,
    out_shape=_nd_shape,
)


def kernel(x, edge_index, W_pre, b_pre, W1, b1, W2, b2, W3, b3, W_head,
           b_head):
    pad_dst = N + (jnp.arange(EPAD, dtype=_I32) % (NPAD - N))
    src2 = jnp.concatenate([edge_index[0], jnp.zeros((EPAD,), _I32)])
    dst2 = jnp.concatenate([edge_index[1], pad_dst])
    src2 = src2.reshape(NW, CHUNKS, K)
    dst2 = dst2.reshape(NW, CHUNKS, K)

    deg0, deg1 = _sc_degree(dst2)
    dis = jax.lax.rsqrt(jnp.maximum(deg0 + deg1, 1.0))
    dis_full = jnp.broadcast_to(dis[:, None], (NPAD, D))

    x_p = jnp.pad(x, ((0, NPAD - N), (0, 0)))
    h, hs = _tc_pre(x_p, W_pre, b_pre.reshape(1, D), dis_full)

    for W, b in ((W1, b1), (W2, b2)):
        aggp = _sc_aggregate(hs, src2, dst2)
        h, hs = _tc_mid(aggp, dis_full, h, W, b.reshape(1, D))

    aggp = _sc_aggregate(hs, src2, dst2)
    out = _tc_final(aggp, dis_full, h, W3, b3.reshape(1, D), W_head,
                    b_head.reshape(1, D))
    return out[:N]


# speedup vs baseline: 2.3216x; 1.2431x over previous
import functools

import jax
import jax.numpy as jnp
from jax import lax
from jax.experimental import pallas as pl
from jax.experimental.pallas import tpu as pltpu
from jax.experimental.pallas import tpu_sc as plsc

N = 10000
E = 320000
D = 128

NC = 2
NS = 16
NW = NC * NS
LANES = 16

K = 80
CHUNKS = 125
EPW = CHUNKS * K
EPAD = NW * EPW - E
NPAD = 10240
ROWS_PS = NPAD // NS
WB = 80
RBLK = 2048
GRID = NPAD // RBLK

_mesh = plsc.VectorSubcoreMesh(core_axis_name="c", subcore_axis_name="s")

_F32 = jnp.float32
_I32 = jnp.int32
_PREC = jax.lax.Precision.HIGHEST


@functools.partial(
    pl.kernel,
    out_type=[jax.ShapeDtypeStruct((NPAD,), _F32),
              jax.ShapeDtypeStruct((NPAD,), _F32)],
    mesh=_mesh,
    scratch_types=[
        pltpu.VMEM_SHARED((NPAD,), _F32),
        pltpu.VMEM((CHUNKS, K), jnp.int32),
        pltpu.VMEM((128,), _F32),
        pltpu.VMEM((ROWS_PS,), _F32),
    ],
)
def _sc_degree(dst_hbm, out0_hbm, out1_hbm, deg_sh, didx, ones_v, zeros_v):
    c = lax.axis_index("c")
    s = lax.axis_index("s")
    wid = s * NC + c

    @pl.loop(0, ROWS_PS // LANES)
    def _(i):
        zeros_v[pl.ds(i * LANES, LANES)] = jnp.zeros((LANES,), _F32)

    @pl.loop(0, 128 // LANES)
    def _(i):
        ones_v[pl.ds(i * LANES, LANES)] = jnp.ones((LANES,), _F32)

    pltpu.sync_copy(zeros_v, deg_sh.at[pl.ds(s * ROWS_PS, ROWS_PS)])
    plsc.subcore_barrier()

    pltpu.sync_copy(dst_hbm.at[wid], didx)

    @pl.loop(0, CHUNKS)
    def _(j):
        pltpu.sync_copy(ones_v.at[pl.ds(0, K)], deg_sh.at[didx.at[j]],
                        add=True)

    plsc.subcore_barrier()

    @pl.when(c == 0)
    def _():
        pltpu.sync_copy(deg_sh.at[pl.ds(s * ROWS_PS, ROWS_PS)],
                        out0_hbm.at[pl.ds(s * ROWS_PS, ROWS_PS)])

    @pl.when(c == 1)
    def _():
        pltpu.sync_copy(deg_sh.at[pl.ds(s * ROWS_PS, ROWS_PS)],
                        out1_hbm.at[pl.ds(s * ROWS_PS, ROWS_PS)])


@functools.partial(
    pl.kernel,
    out_type=jax.ShapeDtypeStruct((NC, NPAD, D), _F32),
    mesh=_mesh,
    scratch_types=[
        pltpu.VMEM_SHARED((NPAD, D), _F32),
        pltpu.VMEM((CHUNKS, K), jnp.int32),
        pltpu.VMEM((CHUNKS, K), jnp.int32),
        pltpu.VMEM((K, D), _F32),
        pltpu.SemaphoreType.DMA,
    ],
)
def _sc_aggregate(hs_hbm, src_hbm, dst_hbm, out_hbm, agg_sh, sidx, didx,
                  rows, sem):
    c = lax.axis_index("c")
    s = lax.axis_index("s")
    wid = s * NC + c

    @pl.loop(0, (WB * D) // LANES)
    def _(t):
        r = t // (D // LANES)
        q = t % (D // LANES)
        rows[r, pl.ds(q * LANES, LANES)] = jnp.zeros((LANES,), _F32)

    @pl.loop(0, ROWS_PS // WB)
    def _(i):
        pltpu.sync_copy(rows.at[pl.ds(0, WB)],
                        agg_sh.at[pl.ds(s * ROWS_PS + i * WB, WB)])

    plsc.subcore_barrier()

    pltpu.sync_copy(src_hbm.at[wid], sidx)
    pltpu.sync_copy(dst_hbm.at[wid], didx)

    @pl.loop(0, CHUNKS)
    def _(j):
        pltpu.async_copy(hs_hbm.at[sidx.at[j]], rows, sem).wait()
        pltpu.sync_copy(rows, agg_sh.at[didx.at[j]], add=True)

    plsc.subcore_barrier()

    @pl.loop(0, ROWS_PS // WB)
    def _(i):
        pltpu.sync_copy(agg_sh.at[pl.ds(s * ROWS_PS + i * WB, WB)],
                        out_hbm.at[c].at[pl.ds(s * ROWS_PS + i * WB, WB)])


def _pre_body(x_ref, w_ref, b_ref, dis_ref, h_ref, hs_ref):
    h = jnp.dot(x_ref[...], w_ref[...], precision=_PREC,
                preferred_element_type=_F32)
    h = jnp.maximum(h + b_ref[...], 0.0)
    h_ref[...] = h
    hs_ref[...] = h * dis_ref[...]


def _mid_body(aggp_ref, dis_ref, h_ref, w_ref, b_ref, hn_ref, hs_ref):
    agg = (aggp_ref[0] + aggp_ref[1]) * dis_ref[...]
    hn = jnp.dot(agg, w_ref[...], precision=_PREC, preferred_element_type=_F32)
    hn = jnp.maximum(hn + b_ref[...] + h_ref[...], 0.0)
    hn_ref[...] = hn
    hs_ref[...] = hn * dis_ref[...]


def _final_body(aggp_ref, dis_ref, h_ref, w_ref, b_ref, wh_ref, bh_ref,
                out_ref):
    agg = (aggp_ref[0] + aggp_ref[1]) * dis_ref[...]
    h3 = jnp.dot(agg, w_ref[...], precision=_PREC, preferred_element_type=_F32)
    h3 = jnp.maximum(h3 + b_ref[...] + h_ref[...], 0.0)
    out_ref[...] = jnp.dot(h3, wh_ref[...], precision=_PREC,
                           preferred_element_type=_F32) + bh_ref[...]


_row_spec = pl.BlockSpec((RBLK, D), lambda i: (i, 0))
_mat_spec = pl.BlockSpec((D, D), lambda i: (0, 0))
_bias_spec = pl.BlockSpec((1, D), lambda i: (0, 0))
_aggp_spec = pl.BlockSpec((NC, RBLK, D), lambda i: (0, i, 0))
_nd_shape = jax.ShapeDtypeStruct((NPAD, D), _F32)


_tc_pre = pl.pallas_call(
    _pre_body,
    grid=(GRID,),
    in_specs=[_row_spec, _mat_spec, _bias_spec, _row_spec],
    out_specs=[_row_spec, _row_spec],
    out_shape=[_nd_shape, _nd_shape],
)

_tc_mid = pl.pallas_call(
    _mid_body,
    grid=(GRID,),
    in_specs=[_aggp_spec, _row_spec, _row_spec, _mat_spec, _bias_spec],
    out_specs=[_row_spec, _row_spec],
    out_shape=[_nd_shape, _nd_shape],
)

_tc_final = pl.pallas_call(
    _final_body,
    grid=(GRID,),
    in_specs=[_aggp_spec, _row_spec, _row_spec, _mat_spec, _bias_spec,
              _mat_spec, _bias_spec],
    out_specs=_row_spec,
    out_shape=_nd_shape,
)


def kernel(x, edge_index, W_pre, b_pre, W1, b1, W2, b2, W3, b3, W_head,
           b_head):
    pad_dst = N + (jnp.arange(EPAD, dtype=_I32) % (NPAD - N))
    src2 = jnp.concatenate([edge_index[0], jnp.zeros((EPAD,), _I32)])
    dst2 = jnp.concatenate([edge_index[1], pad_dst])
    src2 = src2.reshape(NW, CHUNKS, K)
    dst2 = dst2.reshape(NW, CHUNKS, K)

    deg0, deg1 = _sc_degree(dst2)
    dis = jax.lax.rsqrt(jnp.maximum(deg0 + deg1, 1.0))
    dis_full = jnp.broadcast_to(dis[:, None], (NPAD, D))

    x_p = jnp.pad(x, ((0, NPAD - N), (0, 0)))
    h, hs = _tc_pre(x_p, W_pre, b_pre.reshape(1, D), dis_full)

    for W, b in ((W1, b1), (W2, b2)):
        aggp = _sc_aggregate(hs, src2, dst2)
        h, hs = _tc_mid(aggp, dis_full, h, W, b.reshape(1, D))

    aggp = _sc_aggregate(hs, src2, dst2)
    out = _tc_final(aggp, dis_full, h, W3, b3.reshape(1, D), W_head,
                    b_head.reshape(1, D))
    return out[:N]


# speedup vs baseline: 3.6172x; 1.5580x over previous
import functools

import jax
import jax.numpy as jnp
from jax import lax
from jax.experimental import pallas as pl
from jax.experimental.pallas import tpu as pltpu
from jax.experimental.pallas import tpu_sc as plsc

N = 10000
E = 320000
D = 128

NC = 2
NS = 16
NW = NC * NS
LANES = 16

K = 80
CHUNKS = 125
EPW = CHUNKS * K
EPAD = NW * EPW - E
NPAD = 10240
ROWS_PS = NPAD // NS
WB = 80
RBLK = 2048
GRID = NPAD // RBLK

_mesh = plsc.VectorSubcoreMesh(core_axis_name="c", subcore_axis_name="s")

_F32 = jnp.float32
_I32 = jnp.int32
_PREC = jax.lax.Precision.HIGHEST


@functools.partial(
    pl.kernel,
    out_type=[jax.ShapeDtypeStruct((NPAD,), _F32),
              jax.ShapeDtypeStruct((NPAD,), _F32)],
    mesh=_mesh,
    scratch_types=[
        pltpu.VMEM_SHARED((NPAD,), _F32),
        pltpu.VMEM((CHUNKS, K), jnp.int32),
        pltpu.VMEM((128,), _F32),
        pltpu.VMEM((ROWS_PS,), _F32),
    ],
)
def _sc_degree(dst_hbm, out0_hbm, out1_hbm, deg_sh, didx, ones_v, zeros_v):
    c = lax.axis_index("c")
    s = lax.axis_index("s")
    wid = s * NC + c

    @pl.loop(0, ROWS_PS // LANES)
    def _(i):
        zeros_v[pl.ds(i * LANES, LANES)] = jnp.zeros((LANES,), _F32)

    @pl.loop(0, 128 // LANES)
    def _(i):
        ones_v[pl.ds(i * LANES, LANES)] = jnp.ones((LANES,), _F32)

    pltpu.sync_copy(zeros_v, deg_sh.at[pl.ds(s * ROWS_PS, ROWS_PS)])
    plsc.subcore_barrier()

    pltpu.sync_copy(dst_hbm.at[wid], didx)

    @pl.loop(0, CHUNKS)
    def _(j):
        pltpu.sync_copy(ones_v.at[pl.ds(0, K)], deg_sh.at[didx.at[j]],
                        add=True)

    plsc.subcore_barrier()

    @pl.when(c == 0)
    def _():
        pltpu.sync_copy(deg_sh.at[pl.ds(s * ROWS_PS, ROWS_PS)],
                        out0_hbm.at[pl.ds(s * ROWS_PS, ROWS_PS)])

    @pl.when(c == 1)
    def _():
        pltpu.sync_copy(deg_sh.at[pl.ds(s * ROWS_PS, ROWS_PS)],
                        out1_hbm.at[pl.ds(s * ROWS_PS, ROWS_PS)])


@functools.partial(
    pl.kernel,
    out_type=jax.ShapeDtypeStruct((NC, NPAD, D), _F32),
    mesh=_mesh,
    scratch_types=[
        pltpu.VMEM_SHARED((NPAD, D), _F32),
        pltpu.VMEM((EPW,), jnp.int32),
        pltpu.VMEM((CHUNKS, K), jnp.int32),
        pltpu.VMEM((K, D), _F32),
        pltpu.VMEM((K, D), _F32),
        pltpu.SemaphoreType.DMA,
        pltpu.SemaphoreType.DMA,
    ],
)
def _sc_aggregate(hs_hbm, src_hbm, dst_hbm, out_hbm, agg_sh, sidx, didx,
                  rows_a, rows_b, sem_a, sem_b):
    c = lax.axis_index("c")
    s = lax.axis_index("s")
    wid = s * NC + c

    @pl.loop(0, (WB * D) // LANES)
    def _(t):
        r = t // (D // LANES)
        q = t % (D // LANES)
        rows_a[r, pl.ds(q * LANES, LANES)] = jnp.zeros((LANES,), _F32)

    @pl.loop(0, ROWS_PS // WB)
    def _(i):
        pltpu.sync_copy(rows_a.at[pl.ds(0, WB)],
                        agg_sh.at[pl.ds(s * ROWS_PS + i * WB, WB)])

    plsc.subcore_barrier()

    pltpu.sync_copy(src_hbm.at[pl.ds(wid * EPW, EPW)], sidx)
    pltpu.sync_copy(dst_hbm.at[wid], didx)

    def g(j, rows, sem):
        return pltpu.async_copy(hs_hbm.at[sidx.at[pl.ds(j * K, K)]],
                                rows, sem)

    g(0, rows_a, sem_a)

    @pl.loop(0, CHUNKS - 1, step=2)
    def _(j):
        g(j + 1, rows_b, sem_b)
        pltpu.make_async_copy(hs_hbm.at[sidx.at[pl.ds(0, K)]], rows_a,
                              sem_a).wait()
        pltpu.sync_copy(rows_a, agg_sh.at[didx.at[j]], add=True)
        g(j + 2, rows_a, sem_a)
        pltpu.make_async_copy(hs_hbm.at[sidx.at[pl.ds(0, K)]], rows_b,
                              sem_b).wait()
        pltpu.sync_copy(rows_b, agg_sh.at[didx.at[j + 1]], add=True)

    pltpu.make_async_copy(hs_hbm.at[sidx.at[pl.ds(0, K)]], rows_a,
                          sem_a).wait()
    pltpu.sync_copy(rows_a, agg_sh.at[didx.at[CHUNKS - 1]], add=True)

    plsc.subcore_barrier()

    @pl.loop(0, ROWS_PS // WB)
    def _(i):
        pltpu.sync_copy(agg_sh.at[pl.ds(s * ROWS_PS + i * WB, WB)],
                        out_hbm.at[c].at[pl.ds(s * ROWS_PS + i * WB, WB)])


def _pre_body(x_ref, w_ref, b_ref, dis_ref, h_ref, hs_ref):
    h = jnp.dot(x_ref[...], w_ref[...], precision=_PREC,
                preferred_element_type=_F32)
    h = jnp.maximum(h + b_ref[...], 0.0)
    h_ref[...] = h
    hs_ref[...] = h * dis_ref[...]


def _mid_body(aggp_ref, dis_ref, h_ref, w_ref, b_ref, hn_ref, hs_ref):
    agg = (aggp_ref[0] + aggp_ref[1]) * dis_ref[...]
    hn = jnp.dot(agg, w_ref[...], precision=_PREC, preferred_element_type=_F32)
    hn = jnp.maximum(hn + b_ref[...] + h_ref[...], 0.0)
    hn_ref[...] = hn
    hs_ref[...] = hn * dis_ref[...]


def _final_body(aggp_ref, dis_ref, h_ref, w_ref, b_ref, wh_ref, bh_ref,
                out_ref):
    agg = (aggp_ref[0] + aggp_ref[1]) * dis_ref[...]
    h3 = jnp.dot(agg, w_ref[...], precision=_PREC, preferred_element_type=_F32)
    h3 = jnp.maximum(h3 + b_ref[...] + h_ref[...], 0.0)
    out_ref[...] = jnp.dot(h3, wh_ref[...], precision=_PREC,
                           preferred_element_type=_F32) + bh_ref[...]


_row_spec = pl.BlockSpec((RBLK, D), lambda i: (i, 0))
_mat_spec = pl.BlockSpec((D, D), lambda i: (0, 0))
_bias_spec = pl.BlockSpec((1, D), lambda i: (0, 0))
_aggp_spec = pl.BlockSpec((NC, RBLK, D), lambda i: (0, i, 0))
_nd_shape = jax.ShapeDtypeStruct((NPAD, D), _F32)


_tc_pre = pl.pallas_call(
    _pre_body,
    grid=(GRID,),
    in_specs=[_row_spec, _mat_spec, _bias_spec, _row_spec],
    out_specs=[_row_spec, _row_spec],
    out_shape=[_nd_shape, _nd_shape],
)

_tc_mid = pl.pallas_call(
    _mid_body,
    grid=(GRID,),
    in_specs=[_aggp_spec, _row_spec, _row_spec, _mat_spec, _bias_spec],
    out_specs=[_row_spec, _row_spec],
    out_shape=[_nd_shape, _nd_shape],
)

_tc_final = pl.pallas_call(
    _final_body,
    grid=(GRID,),
    in_specs=[_aggp_spec, _row_spec, _row_spec, _mat_spec, _bias_spec,
              _mat_spec, _bias_spec],
    out_specs=_row_spec,
    out_shape=_nd_shape,
)


def kernel(x, edge_index, W_pre, b_pre, W1, b1, W2, b2, W3, b3, W_head,
           b_head):
    pad_dst = N + (jnp.arange(EPAD, dtype=_I32) % (NPAD - N))
    src1 = jnp.concatenate([edge_index[0], jnp.zeros((EPAD,), _I32)])
    dst2 = jnp.concatenate([edge_index[1], pad_dst]).reshape(NW, CHUNKS, K)

    deg0, deg1 = _sc_degree(dst2)
    dis = jax.lax.rsqrt(jnp.maximum(deg0 + deg1, 1.0))
    dis_full = jnp.broadcast_to(dis[:, None], (NPAD, D))

    x_p = jnp.pad(x, ((0, NPAD - N), (0, 0)))
    h, hs = _tc_pre(x_p, W_pre, b_pre.reshape(1, D), dis_full)

    for W, b in ((W1, b1), (W2, b2)):
        aggp = _sc_aggregate(hs, src1, dst2)
        h, hs = _tc_mid(aggp, dis_full, h, W, b.reshape(1, D))

    aggp = _sc_aggregate(hs, src1, dst2)
    out = _tc_final(aggp, dis_full, h, W3, b3.reshape(1, D), W_head,
                    b_head.reshape(1, D))
    return out[:N]
